# Initial kernel scaffold; baseline (speedup 1.0000x reference)
import jax
import jax.numpy as jnp
from jax.experimental import pallas as pl


def kernel(enc, h, r, t, rel_re, rel_im):
    raise NotImplementedError("write your pallas kernel here")


# speedup vs baseline: 15.8927x; 15.8927x over previous
import functools

import jax
import jax.numpy as jnp
from jax import lax
from jax.experimental import pallas as pl
from jax.experimental.pallas import tpu as pltpu
from jax.experimental.pallas import tpu_sc as plsc

NUM_ENTITIES = 1000000
NUM_RELATIONS = 1000
DIM = 64
BATCH = 16384

_info = plsc.get_sparse_core_info()
NC, NS, L = _info.num_cores, _info.num_subcores, _info.num_lanes
NW = NC * NS
B_PER_W = BATCH // NW
CHUNK = 128
N_CHUNKS = B_PER_W // CHUNK


def _buf_types():
  return [
      pltpu.VMEM((CHUNK,), jnp.int32),
      pltpu.VMEM((CHUNK,), jnp.int32),
      pltpu.VMEM((CHUNK,), jnp.int32),
      pltpu.VMEM((CHUNK, 2 * DIM), jnp.float32),
      pltpu.VMEM((CHUNK, 2 * DIM), jnp.float32),
      pltpu.VMEM((CHUNK, DIM), jnp.float32),
      pltpu.VMEM((CHUNK, DIM), jnp.float32),
      pltpu.VMEM((CHUNK,), jnp.float32),
      pltpu.SemaphoreType.DMA,
  ]


def _make_kernel():
  mesh = plsc.VectorSubcoreMesh(core_axis_name="c", subcore_axis_name="s")

  @functools.partial(
      pl.kernel,
      mesh=mesh,
      compiler_params=pltpu.CompilerParams(
          needs_layout_passes=False, use_tc_tiling_on_sc=False),
      out_type=jax.ShapeDtypeStruct((BATCH,), jnp.float32),
      scratch_types=_buf_types() + _buf_types(),
  )
  def scores(enc_h, h_h, r_h, t_h, rre_h, rim_h, out_h, *scratch):
    bufs = (scratch[:9], scratch[9:])
    wid = lax.axis_index("s") * NC + lax.axis_index("c")
    lanes = lax.iota(jnp.int32, L)

    def fire(j):
      ih, it, ir, eh, et, rr, ri, _, sem = bufs[j % 2]
      base = wid * B_PER_W + j * CHUNK
      pltpu.sync_copy(h_h.at[pl.ds(base, CHUNK)], ih)
      pltpu.sync_copy(t_h.at[pl.ds(base, CHUNK)], it)
      pltpu.sync_copy(r_h.at[pl.ds(base, CHUNK)], ir)
      return (pltpu.async_copy(enc_h.at[ih], eh, sem),
              pltpu.async_copy(enc_h.at[it], et, sem),
              pltpu.async_copy(rre_h.at[ir], rr, sem),
              pltpu.async_copy(rim_h.at[ir], ri, sem))

    cps = fire(0)
    for j in range(N_CHUNKS):
      _, _, _, eh_v, et_v, rr_v, ri_v, out_v, _ = bufs[j % 2]
      nxt = fire(j + 1) if j + 1 < N_CHUNKS else None
      for cp in cps:
        cp.wait()

      def group(g, carry):
        gbase = g * L
        out_vec = jnp.zeros((L,), jnp.float32)
        for tt in range(L):
          i = gbase + tt
          acc = jnp.zeros((L,), jnp.float32)
          for c in range(DIM // L):
            lo = c * L
            ehr = eh_v[i, pl.ds(lo, L)]
            ehi = eh_v[i, pl.ds(DIM + lo, L)]
            etr = et_v[i, pl.ds(lo, L)]
            eti = et_v[i, pl.ds(DIM + lo, L)]
            rre = rr_v[i, pl.ds(lo, L)]
            rim = ri_v[i, pl.ds(lo, L)]
            acc = acc + ehr * (rre * etr + rim * eti) + ehi * (rre * eti - rim * etr)
          s = jnp.sum(acc)
          out_vec = out_vec + jnp.where(lanes == tt, s, jnp.float32(0))
        out_v[pl.ds(gbase, L)] = out_vec
        return carry

      lax.fori_loop(0, CHUNK // L, group, 0)
      pltpu.sync_copy(out_v, out_h.at[pl.ds(wid * B_PER_W + j * CHUNK, CHUNK)])
      cps = nxt

  return scores


_scores = _make_kernel()


@jax.jit
def kernel(enc, h, r, t, rel_re, rel_im):
  h = h.astype(jnp.int32)
  r = r.astype(jnp.int32)
  t = t.astype(jnp.int32)
  return _scores(enc, h, r, t, rel_re, rel_im)


# speedup vs baseline: 16.4972x; 1.0380x over previous
import functools

import jax
import jax.numpy as jnp
from jax import lax
from jax.experimental import pallas as pl
from jax.experimental.pallas import tpu as pltpu
from jax.experimental.pallas import tpu_sc as plsc

NUM_ENTITIES = 1000000
NUM_RELATIONS = 1000
DIM = 64
BATCH = 16384

_info = plsc.get_sparse_core_info()
NC, NS, L = _info.num_cores, _info.num_subcores, _info.num_lanes
NW = NC * NS
B_PER_W = BATCH // NW
CHUNK = 128
N_CHUNKS = B_PER_W // CHUNK


def _row_buf_types():
  return [
      pltpu.VMEM((CHUNK, 2 * DIM), jnp.float32),
      pltpu.VMEM((CHUNK, 2 * DIM), jnp.float32),
      pltpu.VMEM((CHUNK, DIM), jnp.float32),
      pltpu.VMEM((CHUNK, DIM), jnp.float32),
      pltpu.VMEM((CHUNK,), jnp.float32),
      pltpu.SemaphoreType.DMA,
  ]


def _make_kernel():
  mesh = plsc.VectorSubcoreMesh(core_axis_name="c", subcore_axis_name="s")

  @functools.partial(
      pl.kernel,
      mesh=mesh,
      compiler_params=pltpu.CompilerParams(
          needs_layout_passes=False, use_tc_tiling_on_sc=False),
      out_type=jax.ShapeDtypeStruct((BATCH,), jnp.float32),
      scratch_types=[
          pltpu.VMEM((N_CHUNKS, CHUNK), jnp.int32),
          pltpu.VMEM((N_CHUNKS, CHUNK), jnp.int32),
          pltpu.VMEM((N_CHUNKS, CHUNK), jnp.int32),
          pltpu.SemaphoreType.DMA,
          pltpu.SemaphoreType.DMA,
          pltpu.SemaphoreType.DMA,
          pltpu.SemaphoreType.DMA,
      ] + _row_buf_types() + _row_buf_types(),
  )
  def scores(enc_h, h_h, r_h, t_h, rre_h, rim_h, out_h,
             idxh_v, idxt_v, idxr_v, si0, si1, si2, si3, *scratch):
    bufs = (scratch[:6], scratch[6:])
    isems = (si0, si1, si2, si3)
    wid = lax.axis_index("s") * NC + lax.axis_index("c")
    lanes = lax.iota(jnp.int32, L)

    icps = []
    for j in range(N_CHUNKS):
      base = wid * B_PER_W + j * CHUNK
      icps.append((
          pltpu.async_copy(h_h.at[pl.ds(base, CHUNK)], idxh_v.at[j], isems[j]),
          pltpu.async_copy(t_h.at[pl.ds(base, CHUNK)], idxt_v.at[j], isems[j]),
          pltpu.async_copy(r_h.at[pl.ds(base, CHUNK)], idxr_v.at[j], isems[j]),
      ))

    def fire(j):
      eh, et, rr, ri, _, sem = bufs[j % 2]
      for cp in icps[j]:
        cp.wait()
      return (pltpu.async_copy(enc_h.at[idxh_v.at[j]], eh, sem),
              pltpu.async_copy(enc_h.at[idxt_v.at[j]], et, sem),
              pltpu.async_copy(rre_h.at[idxr_v.at[j]], rr, sem),
              pltpu.async_copy(rim_h.at[idxr_v.at[j]], ri, sem))

    cps = fire(0)
    for j in range(N_CHUNKS):
      eh_v, et_v, rr_v, ri_v, out_v, _ = bufs[j % 2]
      nxt = fire(j + 1) if j + 1 < N_CHUNKS else None
      for cp in cps:
        cp.wait()

      def group(g, carry):
        gbase = g * L
        out_vec = jnp.zeros((L,), jnp.float32)
        for tt in range(L):
          i = gbase + tt
          acc = jnp.zeros((L,), jnp.float32)
          for c in range(DIM // L):
            lo = c * L
            ehr = eh_v[i, pl.ds(lo, L)]
            ehi = eh_v[i, pl.ds(DIM + lo, L)]
            etr = et_v[i, pl.ds(lo, L)]
            eti = et_v[i, pl.ds(DIM + lo, L)]
            rre = rr_v[i, pl.ds(lo, L)]
            rim = ri_v[i, pl.ds(lo, L)]
            acc = acc + ehr * (rre * etr + rim * eti) + ehi * (rre * eti - rim * etr)
          s = jnp.sum(acc)
          out_vec = out_vec + jnp.where(lanes == tt, s, jnp.float32(0))
        out_v[pl.ds(gbase, L)] = out_vec
        return carry

      lax.fori_loop(0, CHUNK // L, group, 0)
      pltpu.sync_copy(out_v, out_h.at[pl.ds(wid * B_PER_W + j * CHUNK, CHUNK)])
      cps = nxt

  return scores


_scores = _make_kernel()


@jax.jit
def kernel(enc, h, r, t, rel_re, rel_im):
  h = h.astype(jnp.int32)
  r = r.astype(jnp.int32)
  t = t.astype(jnp.int32)
  return _scores(enc, h, r, t, rel_re, rel_im)


# speedup vs baseline: 19.7127x; 1.1949x over previous
import functools

import jax
import jax.numpy as jnp
from jax import lax
from jax.experimental import pallas as pl
from jax.experimental.pallas import tpu as pltpu
from jax.experimental.pallas import tpu_sc as plsc

NUM_ENTITIES = 1000000
NUM_RELATIONS = 1000
DIM = 64
BATCH = 16384

_info = plsc.get_sparse_core_info()
NC, NS, L = _info.num_cores, _info.num_subcores, _info.num_lanes
NW = NC * NS
B_PER_W = BATCH // NW
CHUNK = 128
N_CHUNKS = B_PER_W // CHUNK


def _row_buf_types():
  return [
      pltpu.VMEM((CHUNK, 2 * DIM), jnp.float32),
      pltpu.VMEM((CHUNK, 2 * DIM), jnp.float32),
      pltpu.VMEM((CHUNK, DIM), jnp.float32),
      pltpu.VMEM((CHUNK, DIM), jnp.float32),
      pltpu.VMEM((CHUNK,), jnp.float32),
      pltpu.SemaphoreType.DMA,
  ]


def _make_kernel():
  mesh = plsc.VectorSubcoreMesh(core_axis_name="c", subcore_axis_name="s")

  @functools.partial(
      pl.kernel,
      mesh=mesh,
      compiler_params=pltpu.CompilerParams(
          needs_layout_passes=False, use_tc_tiling_on_sc=False),
      out_type=jax.ShapeDtypeStruct((BATCH,), jnp.float32),
      scratch_types=[
          pltpu.VMEM((N_CHUNKS, CHUNK), jnp.int32),
          pltpu.VMEM((N_CHUNKS, CHUNK), jnp.int32),
          pltpu.VMEM((N_CHUNKS, CHUNK), jnp.int32),
          pltpu.SemaphoreType.DMA,
          pltpu.SemaphoreType.DMA,
          pltpu.SemaphoreType.DMA,
          pltpu.SemaphoreType.DMA,
      ] + _row_buf_types() + _row_buf_types(),
  )
  def scores(enc_h, h_h, r_h, t_h, rre_h, rim_h, out_h,
             idxh_v, idxt_v, idxr_v, si0, si1, si2, si3, *scratch):
    bufs = (scratch[:6], scratch[6:])
    isems = (si0, si1, si2, si3)
    wid = lax.axis_index("s") * NC + lax.axis_index("c")
    lanes = lax.iota(jnp.int32, L)

    icps = []
    for j in range(N_CHUNKS):
      base = wid * B_PER_W + j * CHUNK
      icps.append((
          pltpu.async_copy(h_h.at[pl.ds(base, CHUNK)], idxh_v.at[j], isems[j]),
          pltpu.async_copy(t_h.at[pl.ds(base, CHUNK)], idxt_v.at[j], isems[j]),
          pltpu.async_copy(r_h.at[pl.ds(base, CHUNK)], idxr_v.at[j], isems[j]),
      ))

    def fire(j):
      eh, et, rr, ri, _, sem = bufs[j % 2]
      for cp in icps[j]:
        cp.wait()
      return (pltpu.async_copy(enc_h.at[idxh_v.at[j]], eh, sem),
              pltpu.async_copy(enc_h.at[idxt_v.at[j]], et, sem),
              pltpu.async_copy(rre_h.at[idxr_v.at[j]], rr, sem),
              pltpu.async_copy(rim_h.at[idxr_v.at[j]], ri, sem))

    cps = fire(0)
    for j in range(N_CHUNKS):
      eh_v, et_v, rr_v, ri_v, out_v, _ = bufs[j % 2]
      nxt = fire(j + 1) if j + 1 < N_CHUNKS else None
      for cp in cps:
        cp.wait()

      def group(g, carry):
        gbase = g * L

        @plsc.parallel_loop(0, L, unroll=4, carry=jnp.zeros((L,), jnp.float32))
        def out_vec(tt, ovec):
          i = gbase + tt
          acc = jnp.zeros((L,), jnp.float32)
          for c in range(DIM // L):
            lo = c * L
            ehr = eh_v[i, pl.ds(lo, L)]
            ehi = eh_v[i, pl.ds(DIM + lo, L)]
            etr = et_v[i, pl.ds(lo, L)]
            eti = et_v[i, pl.ds(DIM + lo, L)]
            rre = rr_v[i, pl.ds(lo, L)]
            rim = ri_v[i, pl.ds(lo, L)]
            acc = acc + ehr * (rre * etr + rim * eti) + ehi * (rre * eti - rim * etr)
          s = jnp.sum(acc)
          return ovec + jnp.where(lanes == tt, s, jnp.float32(0))

        out_v[pl.ds(gbase, L)] = out_vec
        return carry

      lax.fori_loop(0, CHUNK // L, group, 0)
      pltpu.sync_copy(out_v, out_h.at[pl.ds(wid * B_PER_W + j * CHUNK, CHUNK)])
      cps = nxt

  return scores


_scores = _make_kernel()


@jax.jit
def kernel(enc, h, r, t, rel_re, rel_im):
  h = h.astype(jnp.int32)
  r = r.astype(jnp.int32)
  t = t.astype(jnp.int32)
  return _scores(enc, h, r, t, rel_re, rel_im)


# speedup vs baseline: 19.7413x; 1.0015x over previous
import functools

import jax
import jax.numpy as jnp
from jax import lax
from jax.experimental import pallas as pl
from jax.experimental.pallas import tpu as pltpu
from jax.experimental.pallas import tpu_sc as plsc

NUM_ENTITIES = 1000000
NUM_RELATIONS = 1000
DIM = 64
BATCH = 16384

_info = plsc.get_sparse_core_info()
NC, NS, L = _info.num_cores, _info.num_subcores, _info.num_lanes
NW = NC * NS
B_PER_W = BATCH // NW
CHUNK = 128
N_CHUNKS = B_PER_W // CHUNK


def _row_buf_types():
  return [
      pltpu.VMEM((CHUNK, 2 * DIM), jnp.float32),
      pltpu.VMEM((CHUNK, 2 * DIM), jnp.float32),
      pltpu.VMEM((CHUNK, DIM), jnp.float32),
      pltpu.VMEM((CHUNK, DIM), jnp.float32),
      pltpu.VMEM((CHUNK,), jnp.float32),
      pltpu.SemaphoreType.DMA,
  ]


def _make_kernel():
  mesh = plsc.VectorSubcoreMesh(core_axis_name="c", subcore_axis_name="s")

  @functools.partial(
      pl.kernel,
      mesh=mesh,
      compiler_params=pltpu.CompilerParams(
          needs_layout_passes=False, use_tc_tiling_on_sc=False),
      out_type=jax.ShapeDtypeStruct((BATCH,), jnp.float32),
      scratch_types=[
          pltpu.VMEM((N_CHUNKS, CHUNK), jnp.int32),
          pltpu.VMEM((N_CHUNKS, CHUNK), jnp.int32),
          pltpu.VMEM((N_CHUNKS, CHUNK), jnp.int32),
          pltpu.SemaphoreType.DMA,
          pltpu.SemaphoreType.DMA,
          pltpu.SemaphoreType.DMA,
          pltpu.SemaphoreType.DMA,
      ] + _row_buf_types() + _row_buf_types(),
  )
  def scores(enc_h, h_h, r_h, t_h, rre_h, rim_h, out_h,
             idxh_v, idxt_v, idxr_v, si0, si1, si2, si3, *scratch):
    bufs = (scratch[:6], scratch[6:])
    isems = (si0, si1, si2, si3)
    wid = lax.axis_index("s") * NC + lax.axis_index("c")
    lanes = lax.iota(jnp.int32, L)

    icps = []
    for j in range(N_CHUNKS):
      base = wid * B_PER_W + j * CHUNK
      icps.append((
          pltpu.async_copy(h_h.at[pl.ds(base, CHUNK)], idxh_v.at[j], isems[j]),
          pltpu.async_copy(t_h.at[pl.ds(base, CHUNK)], idxt_v.at[j], isems[j]),
          pltpu.async_copy(r_h.at[pl.ds(base, CHUNK)], idxr_v.at[j], isems[j]),
      ))

    def fire(j):
      eh, et, rr, ri, _, sem = bufs[j % 2]
      for cp in icps[j]:
        cp.wait()
      return (pltpu.async_copy(enc_h.at[idxh_v.at[j]], eh, sem),
              pltpu.async_copy(enc_h.at[idxt_v.at[j]], et, sem),
              pltpu.async_copy(rre_h.at[idxr_v.at[j]], rr, sem),
              pltpu.async_copy(rim_h.at[idxr_v.at[j]], ri, sem))

    cps = fire(0)
    for j in range(N_CHUNKS):
      eh_v, et_v, rr_v, ri_v, out_v, _ = bufs[j % 2]
      nxt = fire(j + 1) if j + 1 < N_CHUNKS else None
      for cp in cps:
        cp.wait()

      @plsc.parallel_loop(0, CHUNK // L)
      def group(g):
        gbase = g * L

        @plsc.parallel_loop(0, L, unroll=4, carry=jnp.zeros((L,), jnp.float32))
        def out_vec(tt, ovec):
          i = gbase + tt
          acc = jnp.zeros((L,), jnp.float32)
          for c in range(DIM // L):
            lo = c * L
            ehr = eh_v[i, pl.ds(lo, L)]
            ehi = eh_v[i, pl.ds(DIM + lo, L)]
            etr = et_v[i, pl.ds(lo, L)]
            eti = et_v[i, pl.ds(DIM + lo, L)]
            rre = rr_v[i, pl.ds(lo, L)]
            rim = ri_v[i, pl.ds(lo, L)]
            acc = acc + ehr * (rre * etr + rim * eti) + ehi * (rre * eti - rim * etr)
          s = jnp.sum(acc)
          return ovec + jnp.where(lanes == tt, s, jnp.float32(0))

        out_v[pl.ds(gbase, L)] = out_vec

      pltpu.sync_copy(out_v, out_h.at[pl.ds(wid * B_PER_W + j * CHUNK, CHUNK)])
      cps = nxt

  return scores


_scores = _make_kernel()


@jax.jit
def kernel(enc, h, r, t, rel_re, rel_im):
  h = h.astype(jnp.int32)
  r = r.astype(jnp.int32)
  t = t.astype(jnp.int32)
  return _scores(enc, h, r, t, rel_re, rel_im)
